# baseline (speedup 1.0000x reference)
import jax
import jax.numpy as jnp
import numpy as np
from jax.experimental import pallas as pl

_NE = 5
_TK = 3
_NC = 10
_B = 128


def _c1_const():
    c = np.zeros((3, 32, 32), np.float32)
    for j in range(3):
        for g in range(32):
            a, xx1 = divmod(g, 16)
            x = 2 * xx1 + a
            if xx1 <= 12 and x <= 25 and x + j < 32:
                c[j, x + j, g] = 1.0
    return c


def _c2_const():
    c = np.zeros((3, 16, 12), np.float32)
    for j in range(3):
        for g2 in range(12):
            a, xx2 = divmod(g2, 6)
            if xx2 <= 4:
                xx = 2 * xx2 + a + j
                if xx <= 12:
                    c[j, xx, g2] = 1.0
    return c


_C1B = _c1_const()[None, :, :, :, None]
_C2B = _c2_const()[None, :, :, None, :, None]
_B1MASK = np.zeros((1024,), np.float32)
for _g in range(32):
    if _g % 16 <= 12:
        _B1MASK[_g * 32:(_g + 1) * 32] = 1.0
_ONE1 = np.zeros((1024,), np.float32)
_ONE1[511] = 1.0
_B2MASK = np.zeros((768,), np.float32)
for _g in range(12):
    if _g % 6 <= 4:
        _B2MASK[_g * 64:(_g + 1) * 64] = 1.0
_E96 = np.zeros((128, 1), np.float32)
_E96[96, 0] = 1.0
_E511 = np.zeros((1536, 1), np.float32)
_E511[511, 0] = 1.0


def _body(xcat_ref, w1p_ref, w2_ref, wgp_ref, wep_ref, gb_ref, eb_ref,
          out_ref):
    f32 = jnp.float32

    xc = xcat_ref[...]
    c1 = jnp.dot(xc, w1p_ref[...],
                 preferred_element_type=f32)
    p = jnp.maximum(jnp.maximum(c1[:, :512], c1[:, 512:]), 0.0)

    p3 = p.reshape(13, 2, 128, 512)
    q2 = jnp.maximum(p3[:, 0], p3[:, 1]).reshape(1664, 512)
    q2 = q2.astype(jnp.bfloat16)

    w2 = w2_ref[...]
    o2 = jnp.dot(q2[0:1408], w2[0:512], preferred_element_type=f32)
    o2 = o2 + jnp.dot(q2[128:1536], w2[512:1024],
                      preferred_element_type=f32)
    o2 = o2 + jnp.dot(q2[256:1664], w2[1024:1536],
                      preferred_element_type=f32)
    p2 = jnp.maximum(jnp.maximum(o2[:, :384], o2[:, 384:]), 0.0)

    p2r = p2[0:1280].reshape(5, 2, 128, 384)
    h2 = jnp.maximum(p2r[:, 0], p2r[:, 1]).astype(jnp.bfloat16)

    H = jnp.concatenate([h2[0], h2[1], h2[2], h2[3], h2[4]], axis=1)

    g5 = jnp.dot(H, wgp_ref[...], preferred_element_type=f32) + gb_ref[...]
    g5 = g5 - jnp.max(g5, axis=1, keepdims=True)
    eg = jnp.exp(g5)
    gate = eg / jnp.sum(eg, axis=1, keepdims=True)

    ebv = eb_ref[...]
    acc = jnp.zeros((128, _NC), dtype=f32)
    for e in range(_NE):
        ge = gate[:, e:e + 1]
        better = (gate > ge).astype(f32)
        if e > 0:
            tie_lt = (jnp.arange(5) < e).astype(f32)
            better = better + (gate == ge).astype(f32) * tie_lt[None, :]
        rank = jnp.sum(better, axis=1, keepdims=True)
        keep = (rank < float(_TK)).astype(f32)
        eo = jnp.dot(H, wep_ref[e], preferred_element_type=f32)
        eo = eo + ebv[e:e + 1, :]
        acc = acc + keep * ge * eo

    acc = acc - jnp.max(acc, axis=1, keepdims=True)
    ea = jnp.exp(acc)
    out_ref[...] = ea / jnp.sum(ea, axis=1, keepdims=True)


def kernel(inputs, conv1_w, conv1_b, conv2_w, conv2_b, gate_w, gate_b,
           expert_w, expert_b):
    bf16 = jnp.bfloat16

    xp = jnp.pad(inputs[..., 0], ((0, 0), (0, 0), (0, 4)))
    xt = jnp.transpose(xp, (1, 0, 2))
    ones = jnp.ones((26, 128, 32), inputs.dtype)
    xcat = jnp.concatenate([xt[0:26], xt[1:27], xt[2:28], ones],
                           axis=2).reshape(3328, 128).astype(bf16)

    w1s = conv1_w[:, :, 0, :]
    t1 = (_C1B * w1s[:, :, None, None, :]).sum(1)
    b1row = jnp.tile(conv1_b, 32) * _B1MASK + _ONE1
    w1p = (jnp.pad(t1.reshape(96, 1024), ((0, 32), (0, 0)))
           + _E96 * b1row[None, :]).astype(bf16)

    t2 = (_C2B * conv2_w[:, :, None, :, None, :]).sum(1)
    b2row = jnp.tile(conv2_b, 12) * _B2MASK
    w2b = (t2.reshape(1536, 768) + _E511 * b2row[None, :]).astype(bf16)

    wgp = jnp.pad(gate_w.reshape(5, 320, 5),
                  ((0, 0), (0, 64), (0, 0))).reshape(1920, 5).astype(bf16)
    wep = jnp.pad(expert_w.reshape(5, 5, 320, 10),
                  ((0, 0), (0, 0), (0, 64), (0, 0))
                  ).reshape(5, 1920, 10).astype(bf16)

    return pl.pallas_call(
        _body,
        out_shape=jax.ShapeDtypeStruct((_B, _NC), jnp.float32),
    )(xcat, w1p, w2b, wgp, wep, gate_b[None, :], expert_b)


# speedup vs baseline: 2.3713x; 2.3713x over previous
import jax
import jax.numpy as jnp
import numpy as np
from jax.experimental import pallas as pl
from jax.experimental.pallas import tpu as pltpu

_NE = 5
_TK = 3
_NC = 10
_B = 128

_P1 = [(j, g, 2 * (g % 16) + g // 16 + j)
       for j in range(3) for g in range(32)
       if g % 16 <= 12 and 2 * (g % 16) + g // 16 <= 25]

_P2 = [(j, g2, 2 * (g2 % 6) + g2 // 6 + j)
       for j in range(3) for g2 in range(12)
       if g2 % 6 <= 4 and 2 * (g2 % 6) + g2 // 6 + j <= 12]

_OH31 = np.zeros((1, 32), np.float32)
_OH31[0, 31] = 1.0


def _body(x_ref, w1_ref, b1_ref, w2_ref, b2_ref, gw_ref, gb_ref, ew_ref,
          eb_ref, out_ref, w1s, w2s):
    f32 = jnp.float32
    bf16 = jnp.bfloat16

    w1s[...] = jnp.zeros((128, 1024), bf16)
    w1v = w1_ref[...].reshape(9, 32).astype(bf16)
    for i in range(3):
        for (j, g, xx) in _P1:
            w1s[pl.ds(i * 32 + xx, 1), pl.ds(g * 32, 32)] = (
                w1v[3 * i + j: 3 * i + j + 1, :])
    b1v = b1_ref[...].astype(bf16)
    onehot = (jax.lax.broadcasted_iota(jnp.int32, (1, 32), 1)
              == 31).astype(bf16)
    for g in range(32):
        if g % 16 <= 12:
            w1s[pl.ds(96, 1), pl.ds(g * 32, 32)] = b1v
    w1s[pl.ds(96, 1), pl.ds(480, 32)] = onehot
    w1s[pl.ds(96, 1), pl.ds(992, 32)] = onehot

    w2s[...] = jnp.zeros((1536, 768), bf16)
    w2v = w2_ref[...].reshape(288, 64).astype(bf16)
    for i in range(3):
        for (j, g2, xx) in _P2:
            w2s[pl.ds(i * 512 + xx * 32, 32), pl.ds(g2 * 64, 64)] = (
                w2v[(3 * i + j) * 32:(3 * i + j) * 32 + 32, :])
    b2v = b2_ref[...].astype(bf16)
    for g2 in range(12):
        if g2 % 6 <= 4:
            w2s[pl.ds(511, 1), pl.ds(g2 * 64, 64)] = b2v

    xv = x_ref[...]
    xt = jnp.transpose(xv, (1, 0, 2))
    xt = jnp.concatenate([xt, jnp.zeros((28, 128, 4), f32)],
                         axis=2).reshape(3584, 32)
    ones = jnp.ones((3328, 32), f32)
    xc = jnp.concatenate(
        [xt[0:3328], xt[128:3456], xt[256:3584], ones],
        axis=1).astype(bf16)

    c1 = jnp.dot(xc, w1s[...], preferred_element_type=f32)
    p = jnp.maximum(jnp.maximum(c1[:, :512], c1[:, 512:]), 0.0)

    p3 = p.reshape(13, 2, 128, 512)
    q2 = jnp.maximum(p3[:, 0], p3[:, 1]).reshape(1664, 512)
    q2 = q2.astype(bf16)

    o2 = jnp.dot(q2[0:1408], w2s[0:512], preferred_element_type=f32)
    o2 = o2 + jnp.dot(q2[128:1536], w2s[512:1024],
                      preferred_element_type=f32)
    o2 = o2 + jnp.dot(q2[256:1664], w2s[1024:1536],
                      preferred_element_type=f32)
    p2 = jnp.maximum(jnp.maximum(o2[:, :384], o2[:, 384:]), 0.0)

    p2r = p2[0:1280].reshape(5, 2, 128, 384)
    h2 = jnp.maximum(p2r[:, 0], p2r[:, 1]).astype(bf16)

    gw = gw_ref[...].astype(bf16)
    g5 = jnp.dot(h2[0][:, 0:320], gw[0:320],
                 preferred_element_type=f32)
    for k in range(1, 5):
        g5 = g5 + jnp.dot(h2[k][:, 0:320], gw[320 * k:320 * (k + 1)],
                          preferred_element_type=f32)
    g5 = g5 + gb_ref[...]

    g5 = g5 - jnp.max(g5, axis=1, keepdims=True)
    eg = jnp.exp(g5)
    gate = eg / jnp.sum(eg, axis=1, keepdims=True)

    ew = ew_ref[...].astype(bf16)
    ebv = eb_ref[...]
    acc = jnp.zeros((128, _NC), dtype=f32)
    for e in range(_NE):
        ge = gate[:, e:e + 1]
        better = (gate > ge).astype(f32)
        if e > 0:
            tie_lt = (jnp.arange(5) < e).astype(f32)
            better = better + (gate == ge).astype(f32) * tie_lt[None, :]
        rank = jnp.sum(better, axis=1, keepdims=True)
        keep = (rank < float(_TK)).astype(f32)
        eo = jnp.dot(h2[0][:, 0:320], ew[1600 * e:1600 * e + 320],
                     preferred_element_type=f32)
        for k in range(1, 5):
            base = 1600 * e + 320 * k
            eo = eo + jnp.dot(h2[k][:, 0:320], ew[base:base + 320],
                              preferred_element_type=f32)
        eo = eo + ebv[e:e + 1, :]
        acc = acc + keep * ge * eo

    acc = acc - jnp.max(acc, axis=1, keepdims=True)
    ea = jnp.exp(acc)
    out_ref[...] = ea / jnp.sum(ea, axis=1, keepdims=True)


def kernel(inputs, conv1_w, conv1_b, conv2_w, conv2_b, gate_w, gate_b,
           expert_w, expert_b):
    return pl.pallas_call(
        _body,
        out_shape=jax.ShapeDtypeStruct((_B, _NC), jnp.float32),
        scratch_shapes=[
            pltpu.VMEM((128, 1024), jnp.bfloat16),
            pltpu.VMEM((1536, 768), jnp.bfloat16),
        ],
    )(inputs.reshape(128, 28, 28), conv1_w.reshape(9, 32),
      conv1_b.reshape(1, 32), conv2_w.reshape(288, 64),
      conv2_b.reshape(1, 64), gate_w, gate_b.reshape(1, 5),
      expert_w.reshape(8000, 10), expert_b)


# speedup vs baseline: 2.5243x; 1.0645x over previous
import jax
import jax.numpy as jnp
import numpy as np
from jax.experimental import pallas as pl
from jax.experimental.pallas import tpu as pltpu

_NE = 5
_TK = 3
_NC = 10
_B = 128

_P1 = [(j, g, 2 * (g % 16) + g // 16 + j)
       for j in range(3) for g in range(32)
       if g % 16 <= 12 and 2 * (g % 16) + g // 16 <= 25]

_P2 = [(j, g2, 2 * (g2 % 6) + g2 // 6 + j)
       for j in range(3) for g2 in range(12)
       if g2 % 6 <= 4 and 2 * (g2 % 6) + g2 // 6 + j <= 12]

_OH31 = np.zeros((1, 32), np.float32)
_OH31[0, 31] = 1.0


def _body(x_ref, w1_ref, b1_ref, w2_ref, b2_ref, gw_ref, gb_ref, ew_ref,
          eb_ref, out_ref, w1s, w2s):
    f32 = jnp.float32
    bf16 = jnp.bfloat16

    w1s[...] = jnp.zeros((128, 1024), bf16)
    w1v = {(i, j): w1_ref[i, j, 0:1, :].astype(bf16)
           for i in range(3) for j in range(3)}
    for i in range(3):
        for (j, g, xx) in _P1:
            w1s[pl.ds(i * 32 + xx, 1), pl.ds(g * 32, 32)] = w1v[(i, j)]
    b1v = b1_ref[...][None, :].astype(bf16)
    onehot = (jax.lax.broadcasted_iota(jnp.int32, (1, 32), 1)
              == 31).astype(bf16)
    for g in range(32):
        if g % 16 <= 12:
            w1s[pl.ds(96, 1), pl.ds(g * 32, 32)] = b1v
    w1s[pl.ds(96, 1), pl.ds(480, 32)] = onehot
    w1s[pl.ds(96, 1), pl.ds(992, 32)] = onehot

    w2s[...] = jnp.zeros((1536, 768), bf16)
    w2v = {(i, j): w2_ref[i, j, :, :].astype(bf16)
           for i in range(3) for j in range(3)}
    for i in range(3):
        for (j, g2, xx) in _P2:
            w2s[pl.ds(i * 512 + xx * 32, 32), pl.ds(g2 * 64, 64)] = (
                w2v[(i, j)])
    b2v = b2_ref[...][None, :].astype(bf16)
    for g2 in range(12):
        if g2 % 6 <= 4:
            w2s[pl.ds(511, 1), pl.ds(g2 * 64, 64)] = b2v

    xv = x_ref[...]
    xt = jnp.transpose(xv, (1, 0, 2))
    xt = jnp.concatenate([xt, jnp.zeros((28, 128, 4), f32)],
                         axis=2).reshape(3584, 32)
    ones = jnp.ones((3328, 32), f32)
    xc = jnp.concatenate(
        [xt[0:3328], xt[128:3456], xt[256:3584], ones],
        axis=1).astype(bf16)

    c1 = jnp.dot(xc, w1s[...], preferred_element_type=f32)
    p = jnp.maximum(jnp.maximum(c1[:, :512], c1[:, 512:]), 0.0)

    p3 = p.reshape(13, 2, 128, 512)
    q2 = jnp.maximum(p3[:, 0], p3[:, 1]).reshape(1664, 512)
    q2 = q2.astype(bf16)

    o2 = jnp.dot(q2[0:1408], w2s[0:512], preferred_element_type=f32)
    o2 = o2 + jnp.dot(q2[128:1536], w2s[512:1024],
                      preferred_element_type=f32)
    o2 = o2 + jnp.dot(q2[256:1664], w2s[1024:1536],
                      preferred_element_type=f32)
    p2 = jnp.maximum(jnp.maximum(o2[:, :384], o2[:, 384:]), 0.0)

    p2r = p2[0:1280].reshape(5, 2, 128, 384)
    h2 = jnp.maximum(p2r[:, 0], p2r[:, 1]).astype(bf16)

    gw = gw_ref[...].astype(bf16)
    g5 = jnp.dot(h2[0][:, 0:320], gw[0:320],
                 preferred_element_type=f32)
    for k in range(1, 5):
        g5 = g5 + jnp.dot(h2[k][:, 0:320], gw[320 * k:320 * (k + 1)],
                          preferred_element_type=f32)
    g5 = g5 + gb_ref[...][None, :]

    g5 = g5 - jnp.max(g5, axis=1, keepdims=True)
    eg = jnp.exp(g5)
    gate = eg / jnp.sum(eg, axis=1, keepdims=True)

    ebv = eb_ref[...]
    acc = jnp.zeros((128, _NC), dtype=f32)
    for e in range(_NE):
        ge = gate[:, e:e + 1]
        better = (gate > ge).astype(f32)
        if e > 0:
            tie_lt = (jnp.arange(5) < e).astype(f32)
            better = better + (gate == ge).astype(f32) * tie_lt[None, :]
        rank = jnp.sum(better, axis=1, keepdims=True)
        keep = (rank < float(_TK)).astype(f32)
        eo = jnp.dot(h2[0][:, 0:320], ew_ref[e, 0:320, :].astype(bf16),
                     preferred_element_type=f32)
        for k in range(1, 5):
            eo = eo + jnp.dot(h2[k][:, 0:320],
                              ew_ref[e, 320 * k:320 * (k + 1), :].astype(bf16),
                              preferred_element_type=f32)
        eo = eo + ebv[e:e + 1, :]
        acc = acc + keep * ge * eo

    acc = acc - jnp.max(acc, axis=1, keepdims=True)
    ea = jnp.exp(acc)
    out_ref[...] = ea / jnp.sum(ea, axis=1, keepdims=True)


def kernel(inputs, conv1_w, conv1_b, conv2_w, conv2_b, gate_w, gate_b,
           expert_w, expert_b):
    return pl.pallas_call(
        _body,
        out_shape=jax.ShapeDtypeStruct((_B, _NC), jnp.float32),
        scratch_shapes=[
            pltpu.VMEM((128, 1024), jnp.bfloat16),
            pltpu.VMEM((1536, 768), jnp.bfloat16),
        ],
    )(inputs.reshape(128, 28, 28), conv1_w, conv1_b, conv2_w,
      conv2_b, gate_w, gate_b, expert_w, expert_b)


# speedup vs baseline: 2.7996x; 1.1091x over previous
import jax
import jax.numpy as jnp
import numpy as np
from jax.experimental import pallas as pl
from jax.experimental.pallas import tpu as pltpu

_NE = 5
_TK = 3
_NC = 10
_B = 128

_P1 = [(j, g, 2 * (g % 16) + g // 16 + j)
       for j in range(3) for g in range(32)
       if g % 16 <= 12 and 2 * (g % 16) + g // 16 <= 25]

_P2A = [(j, (2 * xx2 + a + j) * 32, a * 128 + xx2 * 64)
        for j in range(3) for a in range(2) for xx2 in range(2)]
_P2B = [(j, (2 * xx2 + a + j - 4) * 32, a * 256 + (xx2 - 2) * 64)
        for j in range(3) for a in range(2) for xx2 in range(2, 5)]

_OH31 = np.zeros((1, 32), np.float32)
_OH31[0, 31] = 1.0


def _body(x_ref, w1_ref, b1_ref, w2_ref, b2_ref, gw_ref, gb_ref, ew_ref,
          eb_ref, out_ref, w1s, w2sa, w2sb):
    f32 = jnp.float32
    bf16 = jnp.bfloat16

    w1s[...] = jnp.zeros((128, 1024), bf16)
    w1v = {(i, j): w1_ref[i, j, 0:1, :].astype(bf16)
           for i in range(3) for j in range(3)}
    for i in range(3):
        for (j, g, xx) in _P1:
            w1s[pl.ds(i * 32 + xx, 1), pl.ds(g * 32, 32)] = w1v[(i, j)]
    b1v = b1_ref[...][None, :].astype(bf16)
    for g in range(32):
        if g % 16 <= 12:
            w1s[pl.ds(96, 1), pl.ds(g * 32, 32)] = b1v

    w2sa[...] = jnp.zeros((768, 256), bf16)
    w2sb[...] = jnp.zeros((768, 512), bf16)
    w2v = {(i, j): w2_ref[i, j, :, :].astype(bf16)
           for i in range(3) for j in range(3)}
    for i in range(3):
        for (j, rb, cb) in _P2A:
            w2sa[pl.ds(i * 256 + rb, 32), pl.ds(cb, 64)] = w2v[(i, j)]
        for (j, rb, cb) in _P2B:
            w2sb[pl.ds(i * 256 + rb, 32), pl.ds(cb, 64)] = w2v[(i, j)]

    xv = x_ref[...]
    xt = jnp.transpose(xv, (1, 0, 2))
    xt = jnp.concatenate([xt, jnp.zeros((28, 128, 4), f32)],
                         axis=2).reshape(3584, 32)
    ones = jnp.ones((3328, 32), f32)
    xc = jnp.concatenate(
        [xt[0:3328], xt[128:3456], xt[256:3584], ones],
        axis=1).astype(bf16)

    c1 = jnp.dot(xc, w1s[...], preferred_element_type=f32)
    p = jnp.maximum(jnp.maximum(c1[:, :512], c1[:, 512:]), 0.0)

    p3 = p.reshape(13, 2, 128, 512)
    q2 = jnp.maximum(p3[:, 0], p3[:, 1]).reshape(1664, 512)
    q2 = q2.astype(bf16)

    ga = jnp.concatenate(
        [q2[0:1408, 0:256], q2[128:1536, 0:256], q2[256:1664, 0:256]],
        axis=1)
    gb = jnp.concatenate(
        [q2[0:1408, 128:384], q2[128:1536, 128:384], q2[256:1664, 128:384]],
        axis=1)
    o2a = jnp.dot(ga, w2sa[...], preferred_element_type=f32)
    o2b = jnp.dot(gb, w2sb[...], preferred_element_type=f32)
    pa = jnp.maximum(o2a[:, 0:128], o2a[:, 128:256])
    pb = jnp.maximum(o2b[:, 0:256], o2b[:, 256:512])
    b2v = b2_ref[...][None, :]
    b2t = jnp.concatenate([b2v] * 6, axis=1)
    p2 = jnp.maximum(jnp.concatenate([pa, pb], axis=1) + b2t, 0.0)

    p2r = p2[0:1280].reshape(5, 2, 128, 384)
    h2 = jnp.maximum(p2r[:, 0], p2r[:, 1]).astype(bf16)

    gw = gw_ref[...].astype(bf16)
    g5 = jnp.dot(h2[0][:, 0:320], gw[0:320],
                 preferred_element_type=f32)
    for k in range(1, 5):
        g5 = g5 + jnp.dot(h2[k][:, 0:320], gw[320 * k:320 * (k + 1)],
                          preferred_element_type=f32)
    g5 = g5 + gb_ref[...][None, :]

    g5 = g5 - jnp.max(g5, axis=1, keepdims=True)
    eg = jnp.exp(g5)
    gate = eg / jnp.sum(eg, axis=1, keepdims=True)

    ebv = eb_ref[...]
    acc = jnp.zeros((128, _NC), dtype=f32)
    for e in range(_NE):
        ge = gate[:, e:e + 1]
        better = (gate > ge).astype(f32)
        if e > 0:
            tie_lt = (jnp.arange(5) < e).astype(f32)
            better = better + (gate == ge).astype(f32) * tie_lt[None, :]
        rank = jnp.sum(better, axis=1, keepdims=True)
        keep = (rank < float(_TK)).astype(f32)
        eo = jnp.dot(h2[0][:, 0:320], ew_ref[e, 0:320, :].astype(bf16),
                     preferred_element_type=f32)
        for k in range(1, 5):
            eo = eo + jnp.dot(h2[k][:, 0:320],
                              ew_ref[e, 320 * k:320 * (k + 1), :].astype(bf16),
                              preferred_element_type=f32)
        eo = eo + ebv[e:e + 1, :]
        acc = acc + keep * ge * eo

    acc = acc - jnp.max(acc, axis=1, keepdims=True)
    ea = jnp.exp(acc)
    out_ref[...] = ea / jnp.sum(ea, axis=1, keepdims=True)


def kernel(inputs, conv1_w, conv1_b, conv2_w, conv2_b, gate_w, gate_b,
           expert_w, expert_b):
    return pl.pallas_call(
        _body,
        out_shape=jax.ShapeDtypeStruct((_B, _NC), jnp.float32),
        scratch_shapes=[
            pltpu.VMEM((128, 1024), jnp.bfloat16),
            pltpu.VMEM((768, 256), jnp.bfloat16),
            pltpu.VMEM((768, 512), jnp.bfloat16),
        ],
    )(inputs.reshape(128, 28, 28), conv1_w, conv1_b, conv2_w,
      conv2_b, gate_w, gate_b, expert_w, expert_b)


# speedup vs baseline: 4.2822x; 1.5295x over previous
import jax
import jax.numpy as jnp
import numpy as np
from jax.experimental import pallas as pl
from jax.experimental.pallas import tpu as pltpu

_NE = 5
_TK = 3
_NC = 10
_B = 128

_P1 = [(j, g, 2 * (g % 16) + g // 16 + j)
       for j in range(3) for g in range(32)
       if g % 16 <= 12 and 2 * (g % 16) + g // 16 <= 25]

_P2A = [(j, (2 * xx2 + a + j) * 32, a * 128 + xx2 * 64)
        for j in range(3) for a in range(2) for xx2 in range(2)]
_P2B = [(j, (2 * xx2 + a + j - 4) * 32, a * 256 + (xx2 - 2) * 64)
        for j in range(3) for a in range(2) for xx2 in range(2, 5)]

_OH31 = np.zeros((1, 32), np.float32)
_OH31[0, 31] = 1.0


def _body(x_ref, w1_ref, b1_ref, w2_ref, b2_ref, wt_ref, gb_ref,
          eb_ref, out_ref, w1s, w2sa, w2sb):
    f32 = jnp.float32
    bf16 = jnp.bfloat16

    w1s[...] = jnp.zeros((128, 1024), bf16)
    w1v = {(i, j): w1_ref[i, j, 0:1, :].astype(bf16)
           for i in range(3) for j in range(3)}
    for i in range(3):
        for (j, g, xx) in _P1:
            w1s[pl.ds(i * 32 + xx, 1), pl.ds(g * 32, 32)] = w1v[(i, j)]
    b1v = b1_ref[...][None, :].astype(bf16)
    for g in range(32):
        if g % 16 <= 12:
            w1s[pl.ds(96, 1), pl.ds(g * 32, 32)] = b1v

    w2sa[...] = jnp.zeros((768, 256), bf16)
    w2sb[...] = jnp.zeros((768, 512), bf16)
    w2v = {(i, j): w2_ref[i, j, :, :].astype(bf16)
           for i in range(3) for j in range(3)}
    for i in range(3):
        for (j, rb, cb) in _P2A:
            w2sa[pl.ds(i * 256 + rb, 32), pl.ds(cb, 64)] = w2v[(i, j)]
        for (j, rb, cb) in _P2B:
            w2sb[pl.ds(i * 256 + rb, 32), pl.ds(cb, 64)] = w2v[(i, j)]

    xc = x_ref[...]

    c1 = jnp.dot(xc, w1s[...], preferred_element_type=f32)
    p = jnp.maximum(jnp.maximum(c1[:, :512], c1[:, 512:]), 0.0)

    p3 = p.reshape(13, 2, 128, 512)
    q2 = jnp.maximum(p3[:, 0], p3[:, 1]).reshape(1664, 512)
    q2 = q2.astype(bf16)

    ga = jnp.concatenate(
        [q2[0:1408, 0:256], q2[128:1536, 0:256], q2[256:1664, 0:256]],
        axis=1)
    gb = jnp.concatenate(
        [q2[0:1408, 128:384], q2[128:1536, 128:384], q2[256:1664, 128:384]],
        axis=1)
    o2a = jnp.dot(ga, w2sa[...], preferred_element_type=f32)
    o2b = jnp.dot(gb, w2sb[...], preferred_element_type=f32)
    pa = jnp.maximum(o2a[:, 0:128], o2a[:, 128:256])
    pb = jnp.maximum(o2b[:, 0:256], o2b[:, 256:512])
    b2v = b2_ref[...][None, :]
    b2t = jnp.concatenate([b2v] * 6, axis=1)
    p2 = jnp.maximum(jnp.concatenate([pa, pb], axis=1) + b2t, 0.0)

    p2r = p2[0:1280].reshape(5, 2, 128, 384)
    h2 = jnp.maximum(p2r[:, 0], p2r[:, 1]).astype(bf16)

    wt = wt_ref[...]
    out55 = jnp.dot(h2[0][:, 0:320], jnp.transpose(wt[:, 0:320], (1, 0)),
                    preferred_element_type=f32)
    for k in range(1, 5):
        out55 = out55 + jnp.dot(
            h2[k][:, 0:320],
            jnp.transpose(wt[:, 320 * k:320 * (k + 1)], (1, 0)),
            preferred_element_type=f32)
    g5 = out55[:, 0:5] + gb_ref[...][None, :]

    g5 = g5 - jnp.max(g5, axis=1, keepdims=True)
    eg = jnp.exp(g5)
    gate = eg / jnp.sum(eg, axis=1, keepdims=True)

    ebv = eb_ref[...]
    acc = jnp.zeros((128, _NC), dtype=f32)
    for e in range(_NE):
        ge = gate[:, e:e + 1]
        better = (gate > ge).astype(f32)
        if e > 0:
            tie_lt = (jnp.arange(5) < e).astype(f32)
            better = better + (gate == ge).astype(f32) * tie_lt[None, :]
        rank = jnp.sum(better, axis=1, keepdims=True)
        keep = (rank < float(_TK)).astype(f32)
        eo = out55[:, 5 + _NC * e: 5 + _NC * (e + 1)] + ebv[e:e + 1, :]
        acc = acc + keep * ge * eo

    acc = acc - jnp.max(acc, axis=1, keepdims=True)
    ea = jnp.exp(acc)
    out_ref[...] = ea / jnp.sum(ea, axis=1, keepdims=True)


def kernel(inputs, conv1_w, conv1_b, conv2_w, conv2_b, gate_w, gate_b,
           expert_w, expert_b):
    bf16 = jnp.bfloat16

    xp = jnp.pad(inputs[..., 0], ((0, 0), (0, 0), (0, 4)))
    xt = jnp.transpose(xp, (1, 0, 2))
    ones = jnp.ones((26, 128, 32), inputs.dtype)
    xcat = jnp.concatenate([xt[0:26], xt[1:27], xt[2:28], ones],
                           axis=2).reshape(3328, 128).astype(bf16)

    wallt = jnp.concatenate(
        [jnp.transpose(gate_w, (1, 0)),
         jnp.transpose(expert_w, (0, 2, 1)).reshape(50, 1600)],
        axis=0).astype(bf16)

    return pl.pallas_call(
        _body,
        out_shape=jax.ShapeDtypeStruct((_B, _NC), jnp.float32),
        scratch_shapes=[
            pltpu.VMEM((128, 1024), jnp.bfloat16),
            pltpu.VMEM((768, 256), jnp.bfloat16),
            pltpu.VMEM((768, 512), jnp.bfloat16),
        ],
    )(xcat, conv1_w, conv1_b, conv2_w, conv2_b, wallt, gate_b, expert_b)
